# baseline (speedup 1.0000x reference)
import jax
import jax.numpy as jnp
from jax import lax
from jax.experimental import pallas as pl
from jax.experimental.pallas import tpu as pltpu
from jax.experimental.pallas import tpu_sc as plsc

NC = 2
NS = 16
NW = NC * NS
LANES = 16

ROW_TILE = 512
F_TILE = 2048
CS_TILE = 128


def _sc_mesh():
    return plsc.VectorSubcoreMesh(
        core_axis_name="c", subcore_axis_name="s", num_cores=NC,
        num_subcores=NS)


def _router_body(x_ref, gw_ref, logits_ref, aux_ref, hist_ref, cnt_ref,
                 csum_ref):
    x = x_ref[...]
    gw = gw_ref[...]
    logits = lax.dot_general(
        x, gw, (((1,), (1,)), ((), ())), preferred_element_type=jnp.float32)
    T, E = logits.shape
    iota_e = lax.broadcasted_iota(jnp.int32, (T, E), 1)
    m1 = jnp.max(logits, axis=1, keepdims=True)
    i1 = jnp.min(jnp.where(logits == m1, iota_e, E), axis=1, keepdims=True)
    masked = jnp.where(iota_e == i1, -jnp.inf, logits)
    m2 = jnp.max(masked, axis=1, keepdims=True)
    i2 = jnp.min(jnp.where(masked == m2, iota_e, E), axis=1, keepdims=True)
    w1 = jax.nn.sigmoid(m1 - m2)
    w2 = 1.0 - w1
    oh1 = (iota_e == i1).astype(jnp.float32)
    oh2 = (iota_e == i2).astype(jnp.float32)
    cnt_ref[...] = oh1 + oh2

    nt = T // CS_TILE
    iota_r = lax.broadcasted_iota(jnp.int32, (CS_TILE, CS_TILE), 0)
    iota_c = lax.broadcasted_iota(jnp.int32, (CS_TILE, CS_TILE), 1)
    tri = (iota_c < iota_r).astype(jnp.float32)

    def body(k, carry):
        c = cnt_ref[pl.ds(k * CS_TILE, CS_TILE), :]
        excl = lax.dot_general(tri, c, (((1,), (0,)), ((), ())),
                               preferred_element_type=jnp.float32)
        csum_ref[pl.ds(k * CS_TILE, CS_TILE), :] = excl + carry
        return carry + jnp.sum(c, axis=0, keepdims=True)

    hist = lax.fori_loop(0, nt, body, jnp.zeros((1, E), jnp.float32))

    ioe_r = lax.broadcasted_iota(jnp.int32, (E, E), 0)
    ioe_c = lax.broadcasted_iota(jnp.int32, (E, E), 1)
    upper = (ioe_r < ioe_c).astype(jnp.float32)
    offs = lax.dot_general(hist, upper, (((1,), (0,)), ((), ())),
                           preferred_element_type=jnp.float32)

    csum = csum_ref[...]
    rank1 = jnp.sum(csum * oh1, axis=1, keepdims=True)
    rank2 = jnp.sum(csum * oh2, axis=1, keepdims=True)
    off_b = jnp.broadcast_to(offs, (T, E))
    off1 = jnp.sum(off_b * oh1, axis=1, keepdims=True)
    off2 = jnp.sum(off_b * oh2, axis=1, keepdims=True)
    dest1 = off1 + rank1
    dest2 = off2 + rank2

    logits_ref[...] = logits
    zero = jnp.zeros_like(w1)
    aux_ref[...] = jnp.concatenate(
        [dest1, dest2, w1, w2, i1.astype(jnp.float32),
         i2.astype(jnp.float32), zero, zero], axis=1)
    hist_ref[...] = hist


def _router(x, gate_w):
    T, _ = x.shape
    E = gate_w.shape[0]
    return pl.pallas_call(
        _router_body,
        out_shape=(
            jax.ShapeDtypeStruct((T, E), jnp.float32),
            jax.ShapeDtypeStruct((T, 8), jnp.float32),
            jax.ShapeDtypeStruct((1, E), jnp.float32),
        ),
        scratch_shapes=[
            pltpu.VMEM((T, E), jnp.float32),
            pltpu.VMEM((T, E), jnp.float32),
        ],
    )(x, gate_w)


def _dispatch_body(x_hbm, d1_hbm, d2_hbm, sx_hbm, i1_v, i2_v, rows_v, sem):
    tpw = rows_v.shape[0]
    wid = lax.axis_index("s") * NC + lax.axis_index("c")
    base = wid * tpw
    pltpu.sync_copy(d1_hbm.at[pl.ds(base, tpw)], i1_v)
    pltpu.sync_copy(d2_hbm.at[pl.ds(base, tpw)], i2_v)
    pltpu.sync_copy(x_hbm.at[pl.ds(base, tpw)], rows_v)
    pltpu.async_copy(rows_v, sx_hbm.at[i1_v], sem).wait()
    pltpu.async_copy(rows_v, sx_hbm.at[i2_v], sem).wait()


def _dispatch(x, d1, d2):
    T, H = x.shape
    R = 2 * T
    tpw = T // NW
    return pl.kernel(
        _dispatch_body,
        out_type=jax.ShapeDtypeStruct((R, H), jnp.float32),
        mesh=_sc_mesh(),
        scratch_types=[
            pltpu.VMEM((tpw,), jnp.int32),
            pltpu.VMEM((tpw,), jnp.int32),
            pltpu.VMEM((tpw, H), jnp.float32),
            pltpu.SemaphoreType.DMA,
        ],
    )(x, d1, d2)


def _gmm_body(tiles, exps, los, his, firsts, x_ref, wl_ref, wv_ref, wo_ref,
              y_ref):
    s = pl.program_id(0)
    f = pl.program_id(1)
    x = x_ref[...].astype(jnp.bfloat16)
    wl = wl_ref[0].astype(jnp.bfloat16)
    wv = wv_ref[0].astype(jnp.bfloat16)
    wo = wo_ref[0].astype(jnp.bfloat16)
    dims = (((1,), (1,)), ((), ()))
    a = lax.dot_general(x, wl, dims, preferred_element_type=jnp.float32)
    b = lax.dot_general(x, wv, dims, preferred_element_type=jnp.float32)
    h = ((a * jax.nn.sigmoid(a)) * b).astype(jnp.bfloat16)
    y = lax.dot_general(h, wo, dims, preferred_element_type=jnp.float32)
    TM = y.shape[0]
    gr = tiles[s] * TM + lax.broadcasted_iota(jnp.int32, (TM, 1), 0)
    valid = jnp.logical_and(gr >= los[s], gr < his[s])
    contrib = jnp.where(valid, y, 0.0)
    is_init = jnp.logical_and(firsts[s] == 1, f == 0)

    @pl.when(is_init)
    def _():
        y_ref[...] = contrib

    @pl.when(jnp.logical_not(is_init))
    def _():
        y_ref[...] += contrib


def _gmm(sorted_x, w_lin, w_v, w_o, meta):
    R, H = sorted_x.shape
    E, F, _ = w_lin.shape
    tiles, exps, los, his, firsts = meta
    s_max = tiles.shape[0]
    nf = F // F_TILE
    TM = ROW_TILE
    grid_spec = pltpu.PrefetchScalarGridSpec(
        num_scalar_prefetch=5,
        grid=(s_max, nf),
        in_specs=[
            pl.BlockSpec((TM, H), lambda s, f, tiles, *_: (tiles[s], 0)),
            pl.BlockSpec((1, F_TILE, H),
                         lambda s, f, tiles, exps, *_: (exps[s], f, 0)),
            pl.BlockSpec((1, F_TILE, H),
                         lambda s, f, tiles, exps, *_: (exps[s], f, 0)),
            pl.BlockSpec((1, H, F_TILE),
                         lambda s, f, tiles, exps, *_: (exps[s], 0, f)),
        ],
        out_specs=pl.BlockSpec((TM, H), lambda s, f, tiles, *_: (tiles[s], 0)),
    )
    return pl.pallas_call(
        _gmm_body,
        grid_spec=grid_spec,
        out_shape=jax.ShapeDtypeStruct((R, H), jnp.float32),
    )(tiles, exps, los, his, firsts, sorted_x, w_lin, w_v, w_o)


def _gmm_metadata(hist_i, R, E):
    TM = ROW_TILE
    g = hist_i
    csum = jnp.cumsum(g)
    o = jnp.concatenate([jnp.zeros((1,), csum.dtype), csum[:-1]])
    nz = g > 0
    ft = o // TM
    lt = jnp.where(nz, (o + g - 1) // TM, ft)
    visits = jnp.where(nz, lt - ft + 1, 0)
    vc = jnp.cumsum(visits)
    s_max = R // TM + E
    s_idx = jnp.arange(s_max, dtype=csum.dtype)
    e_s = jnp.searchsorted(vc, s_idx, side="right")
    pad = s_idx >= vc[-1]
    e_c = jnp.clip(e_s, 0, E - 1)
    vstart = vc[e_c] - visits[e_c]
    tile_s = ft[e_c] + (s_idx - vstart)
    tile_s = jnp.where(pad, R // TM - 1, tile_s)
    lo_s = jnp.where(pad, 0, o[e_c])
    hi_s = jnp.where(pad, 0, o[e_c] + g[e_c])
    first_s = jnp.concatenate(
        [jnp.ones((1,), jnp.bool_), tile_s[1:] != tile_s[:-1]])
    return (tile_s.astype(jnp.int32), e_c.astype(jnp.int32),
            lo_s.astype(jnp.int32), hi_s.astype(jnp.int32),
            first_s.astype(jnp.int32))


def _pair_gather_body(y_hbm, d1_hbm, d2_hbm, y1_hbm, y2_hbm, i1_v, i2_v,
                      b1_v, b2_v, sem):
    tpw = b1_v.shape[0]
    wid = lax.axis_index("s") * NC + lax.axis_index("c")
    base = wid * tpw
    pltpu.sync_copy(d1_hbm.at[pl.ds(base, tpw)], i1_v)
    pltpu.sync_copy(d2_hbm.at[pl.ds(base, tpw)], i2_v)
    pltpu.async_copy(y_hbm.at[i1_v], b1_v, sem).wait()
    pltpu.async_copy(y_hbm.at[i2_v], b2_v, sem).wait()
    pltpu.sync_copy(b1_v, y1_hbm.at[pl.ds(base, tpw)])
    pltpu.sync_copy(b2_v, y2_hbm.at[pl.ds(base, tpw)])


def _pair_gather(y_sorted, d1, d2):
    R, H = y_sorted.shape
    T = d1.shape[0]
    tpw = T // NW
    return pl.kernel(
        _pair_gather_body,
        out_type=(
            jax.ShapeDtypeStruct((T, H), jnp.float32),
            jax.ShapeDtypeStruct((T, H), jnp.float32),
        ),
        mesh=_sc_mesh(),
        scratch_types=[
            pltpu.VMEM((tpw,), jnp.int32),
            pltpu.VMEM((tpw,), jnp.int32),
            pltpu.VMEM((tpw, H), jnp.float32),
            pltpu.VMEM((tpw, H), jnp.float32),
            pltpu.SemaphoreType.DMA,
        ],
    )(y_sorted, d1, d2)


def _epi_body(y1_ref, y2_ref, aux_ref, out_ref):
    w1 = aux_ref[:, 2:3]
    w2 = aux_ref[:, 3:4]
    out_ref[...] = y1_ref[...] * w1 + y2_ref[...] * w2


def _epilogue(y1, y2, aux):
    T, H = y1.shape
    TB = 256
    return pl.pallas_call(
        _epi_body,
        grid=(T // TB,),
        in_specs=[
            pl.BlockSpec((TB, H), lambda i: (i, 0)),
            pl.BlockSpec((TB, H), lambda i: (i, 0)),
            pl.BlockSpec((TB, 8), lambda i: (i, 0)),
        ],
        out_specs=pl.BlockSpec((TB, H), lambda i: (i, 0)),
        out_shape=jax.ShapeDtypeStruct((T, H), jnp.float32),
    )(y1, y2, aux)


@jax.jit
def kernel(hidden_states, gate_w, w_lin, w_v, w_o):
    batch, seq, hdim = hidden_states.shape
    E = gate_w.shape[0]
    x = hidden_states.reshape(-1, hdim)
    T = x.shape[0]
    R = 2 * T

    logits, aux, hist = _router(x, gate_w)
    d1 = aux[:, 0].astype(jnp.int32)
    d2 = aux[:, 1].astype(jnp.int32)
    hist_i = hist[0].astype(jnp.int32)

    sorted_x = _dispatch(x, d1, d2)
    meta = _gmm_metadata(hist_i, R, E)
    y_sorted = _gmm(sorted_x, w_lin, w_v, w_o, meta)
    y1, y2 = _pair_gather(y_sorted, d1, d2)
    out = _epilogue(y1, y2, aux)
    return out.reshape(batch, seq, hdim), logits


# speedup vs baseline: 1.0115x; 1.0115x over previous
import jax
import jax.numpy as jnp
from jax import lax
from jax.experimental import pallas as pl
from jax.experimental.pallas import tpu as pltpu
from jax.experimental.pallas import tpu_sc as plsc

NC = 2
NS = 16
NW = NC * NS
LANES = 16
WS_W = 128

ROW_TILE = 256
F_TILE = 2048
CS_TILE = 128


def _sc_mesh():
    return plsc.VectorSubcoreMesh(
        core_axis_name="c", subcore_axis_name="s", num_cores=NC,
        num_subcores=NS)


def _router_body(x_ref, gw_ref, logits_ref, aux_ref, hist_ref, w1r_ref,
                 w2r_ref, cnt_ref, csum_ref):
    x = x_ref[...]
    gw = gw_ref[...]
    logits = lax.dot_general(
        x, gw, (((1,), (1,)), ((), ())), preferred_element_type=jnp.float32)
    T, E = logits.shape
    iota_e = lax.broadcasted_iota(jnp.int32, (T, E), 1)
    m1 = jnp.max(logits, axis=1, keepdims=True)
    i1 = jnp.min(jnp.where(logits == m1, iota_e, E), axis=1, keepdims=True)
    masked = jnp.where(iota_e == i1, -jnp.inf, logits)
    m2 = jnp.max(masked, axis=1, keepdims=True)
    i2 = jnp.min(jnp.where(masked == m2, iota_e, E), axis=1, keepdims=True)
    w1 = jax.nn.sigmoid(m1 - m2)
    w2 = 1.0 - w1
    oh1 = (iota_e == i1).astype(jnp.float32)
    oh2 = (iota_e == i2).astype(jnp.float32)
    cnt_ref[...] = oh1 + oh2

    nt = T // CS_TILE
    iota_r = lax.broadcasted_iota(jnp.int32, (CS_TILE, CS_TILE), 0)
    iota_c = lax.broadcasted_iota(jnp.int32, (CS_TILE, CS_TILE), 1)
    tri = (iota_c < iota_r).astype(jnp.float32)

    def body(k, carry):
        c = cnt_ref[pl.ds(k * CS_TILE, CS_TILE), :]
        excl = lax.dot_general(tri, c, (((1,), (0,)), ((), ())),
                               preferred_element_type=jnp.float32)
        csum_ref[pl.ds(k * CS_TILE, CS_TILE), :] = excl + carry
        return carry + jnp.sum(c, axis=0, keepdims=True)

    hist = lax.fori_loop(0, nt, body, jnp.zeros((1, E), jnp.float32))

    ioe_r = lax.broadcasted_iota(jnp.int32, (E, E), 0)
    ioe_c = lax.broadcasted_iota(jnp.int32, (E, E), 1)
    upper = (ioe_r < ioe_c).astype(jnp.float32)
    offs = lax.dot_general(hist, upper, (((1,), (0,)), ((), ())),
                           preferred_element_type=jnp.float32)

    csum = csum_ref[...]
    rank1 = jnp.sum(csum * oh1, axis=1, keepdims=True)
    rank2 = jnp.sum(csum * oh2, axis=1, keepdims=True)
    off_b = jnp.broadcast_to(offs, (T, E))
    off1 = jnp.sum(off_b * oh1, axis=1, keepdims=True)
    off2 = jnp.sum(off_b * oh2, axis=1, keepdims=True)
    dest1 = off1 + rank1
    dest2 = off2 + rank2

    logits_ref[...] = logits
    zero = jnp.zeros_like(w1)
    aux_ref[...] = jnp.concatenate(
        [dest1, dest2, w1, w2, i1.astype(jnp.float32),
         i2.astype(jnp.float32), zero, zero], axis=1)
    hist_ref[...] = hist
    w1r_ref[...] = jnp.broadcast_to(w1, (T, WS_W))
    w2r_ref[...] = jnp.broadcast_to(w2, (T, WS_W))


def _router(x, gate_w):
    T, _ = x.shape
    E = gate_w.shape[0]
    return pl.pallas_call(
        _router_body,
        out_shape=(
            jax.ShapeDtypeStruct((T, E), jnp.float32),
            jax.ShapeDtypeStruct((T, 8), jnp.float32),
            jax.ShapeDtypeStruct((1, E), jnp.float32),
            jax.ShapeDtypeStruct((T, WS_W), jnp.float32),
            jax.ShapeDtypeStruct((T, WS_W), jnp.float32),
        ),
        scratch_shapes=[
            pltpu.VMEM((T, E), jnp.float32),
            pltpu.VMEM((T, E), jnp.float32),
        ],
    )(x, gate_w)


def _dispatch_body(x_hbm, d1_hbm, d2_hbm, w1r_hbm, w2r_hbm, sx_hbm, wsr_hbm,
                   i1_v, i2_v, rows_v, w1b_v, w2b_v, sem):
    tpw = rows_v.shape[0]
    wid = lax.axis_index("s") * NC + lax.axis_index("c")
    base = wid * tpw
    pltpu.sync_copy(d1_hbm.at[pl.ds(base, tpw)], i1_v)
    pltpu.sync_copy(d2_hbm.at[pl.ds(base, tpw)], i2_v)
    pltpu.sync_copy(x_hbm.at[pl.ds(base, tpw)], rows_v)
    pltpu.sync_copy(w1r_hbm.at[pl.ds(base, tpw)], w1b_v)
    pltpu.sync_copy(w2r_hbm.at[pl.ds(base, tpw)], w2b_v)
    c1 = pltpu.async_copy(rows_v, sx_hbm.at[i1_v], sem)
    c2 = pltpu.async_copy(rows_v, sx_hbm.at[i2_v], sem)
    c3 = pltpu.async_copy(w1b_v, wsr_hbm.at[i1_v], sem)
    c4 = pltpu.async_copy(w2b_v, wsr_hbm.at[i2_v], sem)
    c1.wait()
    c2.wait()
    c3.wait()
    c4.wait()


def _dispatch(x, d1, d2, w1r, w2r):
    T, H = x.shape
    R = 2 * T
    tpw = T // NW
    return pl.kernel(
        _dispatch_body,
        out_type=(
            jax.ShapeDtypeStruct((R, H), jnp.float32),
            jax.ShapeDtypeStruct((R, WS_W), jnp.float32),
        ),
        mesh=_sc_mesh(),
        scratch_types=[
            pltpu.VMEM((tpw,), jnp.int32),
            pltpu.VMEM((tpw,), jnp.int32),
            pltpu.VMEM((tpw, H), jnp.float32),
            pltpu.VMEM((tpw, WS_W), jnp.float32),
            pltpu.VMEM((tpw, WS_W), jnp.float32),
            pltpu.SemaphoreType.DMA,
        ],
    )(x, d1, d2, w1r, w2r)


def _gmm_body(tiles, exps, los, his, firsts, x_ref, wl_ref, wv_ref, wo_ref,
              ws_ref, y_ref):
    s = pl.program_id(0)
    f = pl.program_id(1)
    x = x_ref[...]
    wl = wl_ref[0]
    wv = wv_ref[0]
    wo = wo_ref[0]
    dims = (((1,), (1,)), ((), ()))
    a = lax.dot_general(x, wl, dims, preferred_element_type=jnp.float32)
    b = lax.dot_general(x, wv, dims, preferred_element_type=jnp.float32)
    h = (a * jax.nn.sigmoid(a)) * b
    y = lax.dot_general(h, wo, dims, preferred_element_type=jnp.float32)
    TM = y.shape[0]
    gr = tiles[s] * TM + lax.broadcasted_iota(jnp.int32, (TM, 1), 0)
    valid = jnp.logical_and(gr >= los[s], gr < his[s])
    contrib = jnp.where(valid, y * ws_ref[:, 0:1], 0.0)
    is_init = jnp.logical_and(firsts[s] == 1, f == 0)

    @pl.when(is_init)
    def _():
        y_ref[...] = contrib

    @pl.when(jnp.logical_not(is_init))
    def _():
        y_ref[...] += contrib


def _gmm(sorted_x, w_lin, w_v, w_o, wsr, meta):
    R, H = sorted_x.shape
    E, F, _ = w_lin.shape
    tiles, exps, los, his, firsts = meta
    s_max = tiles.shape[0]
    nf = F // F_TILE
    TM = ROW_TILE
    grid_spec = pltpu.PrefetchScalarGridSpec(
        num_scalar_prefetch=5,
        grid=(s_max, nf),
        in_specs=[
            pl.BlockSpec((TM, H), lambda s, f, tiles, *_: (tiles[s], 0)),
            pl.BlockSpec((1, F_TILE, H),
                         lambda s, f, tiles, exps, *_: (exps[s], f, 0)),
            pl.BlockSpec((1, F_TILE, H),
                         lambda s, f, tiles, exps, *_: (exps[s], f, 0)),
            pl.BlockSpec((1, H, F_TILE),
                         lambda s, f, tiles, exps, *_: (exps[s], 0, f)),
            pl.BlockSpec((TM, WS_W), lambda s, f, tiles, *_: (tiles[s], 0)),
        ],
        out_specs=pl.BlockSpec((TM, H), lambda s, f, tiles, *_: (tiles[s], 0)),
    )
    return pl.pallas_call(
        _gmm_body,
        grid_spec=grid_spec,
        out_shape=jax.ShapeDtypeStruct((R, H), jnp.float32),
    )(tiles, exps, los, his, firsts, sorted_x, w_lin, w_v, w_o, wsr)


def _gmm_metadata(hist_i, R, E):
    TM = ROW_TILE
    g = hist_i
    csum = jnp.cumsum(g)
    o = jnp.concatenate([jnp.zeros((1,), csum.dtype), csum[:-1]])
    nz = g > 0
    ft = o // TM
    lt = jnp.where(nz, (o + g - 1) // TM, ft)
    visits = jnp.where(nz, lt - ft + 1, 0)
    vc = jnp.cumsum(visits)
    s_max = R // TM + E
    s_idx = jnp.arange(s_max, dtype=csum.dtype)
    e_s = jnp.searchsorted(vc, s_idx, side="right")
    pad = s_idx >= vc[-1]
    e_c = jnp.clip(e_s, 0, E - 1)
    vstart = vc[e_c] - visits[e_c]
    tile_s = ft[e_c] + (s_idx - vstart)
    tile_s = jnp.where(pad, R // TM - 1, tile_s)
    lo_s = jnp.where(pad, 0, o[e_c])
    hi_s = jnp.where(pad, 0, o[e_c] + g[e_c])
    first_s = jnp.concatenate(
        [jnp.ones((1,), jnp.bool_), tile_s[1:] != tile_s[:-1]])
    return (tile_s.astype(jnp.int32), e_c.astype(jnp.int32),
            lo_s.astype(jnp.int32), hi_s.astype(jnp.int32),
            first_s.astype(jnp.int32))


def _combine_body(y_hbm, d1_hbm, d2_hbm, out_hbm, i1_v, i2_v, b1_v, b2_v,
                  sem):
    tpw, H = b1_v.shape
    wid = lax.axis_index("s") * NC + lax.axis_index("c")
    base = wid * tpw
    pltpu.sync_copy(d1_hbm.at[pl.ds(base, tpw)], i1_v)
    pltpu.sync_copy(d2_hbm.at[pl.ds(base, tpw)], i2_v)
    c1 = pltpu.async_copy(y_hbm.at[i1_v], b1_v, sem)
    c2 = pltpu.async_copy(y_hbm.at[i2_v], b2_v, sem)
    c1.wait()
    c2.wait()
    npc = H // LANES

    def body(i, _):
        r = i // npc
        c = (i % npc) * LANES
        b1_v[r, pl.ds(c, LANES)] += b2_v[r, pl.ds(c, LANES)]
        return 0

    lax.fori_loop(0, tpw * npc, body, 0)
    pltpu.sync_copy(b1_v, out_hbm.at[pl.ds(base, tpw)])


def _combine(y_sorted, d1, d2):
    R, H = y_sorted.shape
    T = d1.shape[0]
    tpw = T // NW
    return pl.kernel(
        _combine_body,
        out_type=jax.ShapeDtypeStruct((T, H), jnp.float32),
        mesh=_sc_mesh(),
        scratch_types=[
            pltpu.VMEM((tpw,), jnp.int32),
            pltpu.VMEM((tpw,), jnp.int32),
            pltpu.VMEM((tpw, H), jnp.float32),
            pltpu.VMEM((tpw, H), jnp.float32),
            pltpu.SemaphoreType.DMA,
        ],
    )(y_sorted, d1, d2)


@jax.jit
def kernel(hidden_states, gate_w, w_lin, w_v, w_o):
    batch, seq, hdim = hidden_states.shape
    E = gate_w.shape[0]
    x = hidden_states.reshape(-1, hdim)
    T = x.shape[0]
    R = 2 * T

    logits, aux, hist, w1r, w2r = _router(x, gate_w)
    d1 = aux[:, 0].astype(jnp.int32)
    d2 = aux[:, 1].astype(jnp.int32)
    hist_i = hist[0].astype(jnp.int32)

    sorted_x, wsr = _dispatch(x, d1, d2, w1r, w2r)
    meta = _gmm_metadata(hist_i, R, E)
    y_sorted = _gmm(sorted_x, w_lin, w_v, w_o, wsr, meta)
    out = _combine(y_sorted, d1, d2)
    return out.reshape(batch, seq, hdim), logits


# speedup vs baseline: 1.0169x; 1.0054x over previous
import jax
import jax.numpy as jnp
from jax import lax
from jax.experimental import pallas as pl
from jax.experimental.pallas import tpu as pltpu
from jax.experimental.pallas import tpu_sc as plsc

NC = 2
NS = 16
NW = NC * NS
LANES = 16
WS_W = 128

ROW_TILE = 256
F_TILE = 2048
CS_TILE = 128


def _sc_mesh():
    return plsc.VectorSubcoreMesh(
        core_axis_name="c", subcore_axis_name="s", num_cores=NC,
        num_subcores=NS)


def _router_body(x_ref, gw_ref, logits_ref, aux_ref, hist_ref, w1r_ref,
                 w2r_ref, cnt_ref, csum_ref):
    x = x_ref[...]
    gw = gw_ref[...]
    logits = lax.dot_general(
        x, gw, (((1,), (1,)), ((), ())), preferred_element_type=jnp.float32)
    T, E = logits.shape
    iota_e = lax.broadcasted_iota(jnp.int32, (T, E), 1)
    m1 = jnp.max(logits, axis=1, keepdims=True)
    i1 = jnp.min(jnp.where(logits == m1, iota_e, E), axis=1, keepdims=True)
    masked = jnp.where(iota_e == i1, -jnp.inf, logits)
    m2 = jnp.max(masked, axis=1, keepdims=True)
    i2 = jnp.min(jnp.where(masked == m2, iota_e, E), axis=1, keepdims=True)
    w1 = jax.nn.sigmoid(m1 - m2)
    w2 = 1.0 - w1
    oh1 = (iota_e == i1).astype(jnp.float32)
    oh2 = (iota_e == i2).astype(jnp.float32)
    cnt_ref[...] = oh1 + oh2

    nt = T // CS_TILE
    iota_r = lax.broadcasted_iota(jnp.int32, (CS_TILE, CS_TILE), 0)
    iota_c = lax.broadcasted_iota(jnp.int32, (CS_TILE, CS_TILE), 1)
    tri = (iota_c < iota_r).astype(jnp.float32)

    def body(k, carry):
        c = cnt_ref[pl.ds(k * CS_TILE, CS_TILE), :]
        excl = lax.dot_general(tri, c, (((1,), (0,)), ((), ())),
                               preferred_element_type=jnp.float32)
        csum_ref[pl.ds(k * CS_TILE, CS_TILE), :] = excl + carry
        return carry + jnp.sum(c, axis=0, keepdims=True)

    hist = lax.fori_loop(0, nt, body, jnp.zeros((1, E), jnp.float32))

    ioe_r = lax.broadcasted_iota(jnp.int32, (E, E), 0)
    ioe_c = lax.broadcasted_iota(jnp.int32, (E, E), 1)
    upper = (ioe_r < ioe_c).astype(jnp.float32)
    tiles_e = jnp.floor((hist + (ROW_TILE - 1)) * (1.0 / ROW_TILE))
    offs = ROW_TILE * lax.dot_general(
        tiles_e, upper, (((1,), (0,)), ((), ())),
        preferred_element_type=jnp.float32)

    csum = csum_ref[...]
    rank1 = jnp.sum(csum * oh1, axis=1, keepdims=True)
    rank2 = jnp.sum(csum * oh2, axis=1, keepdims=True)
    off_b = jnp.broadcast_to(offs, (T, E))
    off1 = jnp.sum(off_b * oh1, axis=1, keepdims=True)
    off2 = jnp.sum(off_b * oh2, axis=1, keepdims=True)
    dest1 = off1 + rank1
    dest2 = off2 + rank2

    logits_ref[...] = logits
    zero = jnp.zeros_like(w1)
    aux_ref[...] = jnp.concatenate(
        [dest1, dest2, w1, w2, i1.astype(jnp.float32),
         i2.astype(jnp.float32), zero, zero], axis=1)
    hist_ref[...] = hist
    w1r_ref[...] = jnp.broadcast_to(w1, (T, WS_W))
    w2r_ref[...] = jnp.broadcast_to(w2, (T, WS_W))


def _router(x, gate_w):
    T, _ = x.shape
    E = gate_w.shape[0]
    return pl.pallas_call(
        _router_body,
        out_shape=(
            jax.ShapeDtypeStruct((T, E), jnp.float32),
            jax.ShapeDtypeStruct((T, 8), jnp.float32),
            jax.ShapeDtypeStruct((1, E), jnp.float32),
            jax.ShapeDtypeStruct((T, WS_W), jnp.float32),
            jax.ShapeDtypeStruct((T, WS_W), jnp.float32),
        ),
        scratch_shapes=[
            pltpu.VMEM((T, E), jnp.float32),
            pltpu.VMEM((T, E), jnp.float32),
        ],
    )(x, gate_w)


def _dispatch_body(x_hbm, d1_hbm, d2_hbm, w1r_hbm, w2r_hbm, sx_hbm, wsr_hbm,
                   i1_v, i2_v, rows_v, w1b_v, w2b_v, sem):
    tpw = rows_v.shape[0]
    wid = lax.axis_index("s") * NC + lax.axis_index("c")
    base = wid * tpw
    pltpu.sync_copy(d1_hbm.at[pl.ds(base, tpw)], i1_v)
    pltpu.sync_copy(d2_hbm.at[pl.ds(base, tpw)], i2_v)
    pltpu.sync_copy(x_hbm.at[pl.ds(base, tpw)], rows_v)
    pltpu.sync_copy(w1r_hbm.at[pl.ds(base, tpw)], w1b_v)
    pltpu.sync_copy(w2r_hbm.at[pl.ds(base, tpw)], w2b_v)
    c1 = pltpu.async_copy(rows_v, sx_hbm.at[i1_v], sem)
    c2 = pltpu.async_copy(rows_v, sx_hbm.at[i2_v], sem)
    c3 = pltpu.async_copy(w1b_v, wsr_hbm.at[i1_v], sem)
    c4 = pltpu.async_copy(w2b_v, wsr_hbm.at[i2_v], sem)
    c1.wait()
    c2.wait()
    c3.wait()
    c4.wait()


def _dispatch(x, d1, d2, w1r, w2r, r_pad):
    T, H = x.shape
    tpw = T // NW
    return pl.kernel(
        _dispatch_body,
        out_type=(
            jax.ShapeDtypeStruct((r_pad, H), jnp.float32),
            jax.ShapeDtypeStruct((r_pad, WS_W), jnp.float32),
        ),
        mesh=_sc_mesh(),
        scratch_types=[
            pltpu.VMEM((tpw,), jnp.int32),
            pltpu.VMEM((tpw,), jnp.int32),
            pltpu.VMEM((tpw, H), jnp.float32),
            pltpu.VMEM((tpw, WS_W), jnp.float32),
            pltpu.VMEM((tpw, WS_W), jnp.float32),
            pltpu.SemaphoreType.DMA,
        ],
    )(x, d1, d2, w1r, w2r)


def _gmm_body(tiles, exps, los, his, firsts, x_ref, wl_ref, wv_ref, wo_ref,
              ws_ref, y_ref):
    s = pl.program_id(0)
    f = pl.program_id(1)
    x = x_ref[...]
    wl = wl_ref[0]
    wv = wv_ref[0]
    wo = wo_ref[0]
    dims = (((1,), (1,)), ((), ()))
    a = lax.dot_general(x, wl, dims, preferred_element_type=jnp.float32)
    b = lax.dot_general(x, wv, dims, preferred_element_type=jnp.float32)
    h = (a * jax.nn.sigmoid(a)) * b
    y = lax.dot_general(h, wo, dims, preferred_element_type=jnp.float32)
    TM = y.shape[0]
    gr = tiles[s] * TM + lax.broadcasted_iota(jnp.int32, (TM, 1), 0)
    valid = jnp.logical_and(gr >= los[s], gr < his[s])
    contrib = jnp.where(valid, y * ws_ref[:, 0:1], 0.0)
    is_init = jnp.logical_and(firsts[s] == 1, f == 0)

    @pl.when(is_init)
    def _():
        y_ref[...] = contrib

    @pl.when(jnp.logical_not(is_init))
    def _():
        y_ref[...] += contrib


def _gmm(sorted_x, w_lin, w_v, w_o, wsr, meta):
    R, H = sorted_x.shape
    E, F, _ = w_lin.shape
    tiles, exps, los, his, firsts = meta
    s_max = tiles.shape[0]
    nf = F // F_TILE
    TM = ROW_TILE
    grid_spec = pltpu.PrefetchScalarGridSpec(
        num_scalar_prefetch=5,
        grid=(s_max, nf),
        in_specs=[
            pl.BlockSpec((TM, H), lambda s, f, tiles, *_: (tiles[s], 0)),
            pl.BlockSpec((1, F_TILE, H),
                         lambda s, f, tiles, exps, *_: (exps[s], f, 0)),
            pl.BlockSpec((1, F_TILE, H),
                         lambda s, f, tiles, exps, *_: (exps[s], f, 0)),
            pl.BlockSpec((1, H, F_TILE),
                         lambda s, f, tiles, exps, *_: (exps[s], 0, f)),
            pl.BlockSpec((TM, WS_W), lambda s, f, tiles, *_: (tiles[s], 0)),
        ],
        out_specs=pl.BlockSpec((TM, H), lambda s, f, tiles, *_: (tiles[s], 0)),
    )
    return pl.pallas_call(
        _gmm_body,
        grid_spec=grid_spec,
        out_shape=jax.ShapeDtypeStruct((R, H), jnp.float32),
    )(tiles, exps, los, his, firsts, sorted_x, w_lin, w_v, w_o, wsr)


def _gmm_metadata(hist_i, R, E):
    TM = ROW_TILE
    g = hist_i
    gt = (g + TM - 1) // TM
    vc = jnp.cumsum(gt)
    o_pad = TM * jnp.concatenate([jnp.zeros((1,), vc.dtype), vc[:-1]])
    s_max = R // TM + E
    s_idx = jnp.arange(s_max, dtype=vc.dtype)
    e_s = jnp.searchsorted(vc, s_idx, side="right")
    v_total = vc[-1]
    pad = s_idx >= v_total
    e_c = jnp.clip(e_s, 0, E - 1)
    tile_s = jnp.clip(s_idx, 0, v_total - 1)
    lo_s = jnp.where(pad, 0, o_pad[e_c])
    hi_s = jnp.where(pad, 0, o_pad[e_c] + g[e_c])
    first_s = jnp.logical_not(pad)
    return (tile_s.astype(jnp.int32), e_c.astype(jnp.int32),
            lo_s.astype(jnp.int32), hi_s.astype(jnp.int32),
            first_s.astype(jnp.int32))


def _combine_body(y_hbm, d1_hbm, d2_hbm, out_hbm, i1_v, i2_v, b1_v, b2_v,
                  sem):
    tpw, H = b1_v.shape
    wid = lax.axis_index("s") * NC + lax.axis_index("c")
    base = wid * tpw
    pltpu.sync_copy(d1_hbm.at[pl.ds(base, tpw)], i1_v)
    pltpu.sync_copy(d2_hbm.at[pl.ds(base, tpw)], i2_v)
    c1 = pltpu.async_copy(y_hbm.at[i1_v], b1_v, sem)
    c2 = pltpu.async_copy(y_hbm.at[i2_v], b2_v, sem)
    c1.wait()
    c2.wait()
    npc = H // LANES

    def body(i, _):
        r = i // npc
        c = (i % npc) * LANES
        b1_v[r, pl.ds(c, LANES)] += b2_v[r, pl.ds(c, LANES)]
        return 0

    lax.fori_loop(0, tpw * npc, body, 0)
    pltpu.sync_copy(b1_v, out_hbm.at[pl.ds(base, tpw)])


def _combine(y_sorted, d1, d2):
    R, H = y_sorted.shape
    T = d1.shape[0]
    tpw = T // NW
    return pl.kernel(
        _combine_body,
        out_type=jax.ShapeDtypeStruct((T, H), jnp.float32),
        mesh=_sc_mesh(),
        scratch_types=[
            pltpu.VMEM((tpw,), jnp.int32),
            pltpu.VMEM((tpw,), jnp.int32),
            pltpu.VMEM((tpw, H), jnp.float32),
            pltpu.VMEM((tpw, H), jnp.float32),
            pltpu.SemaphoreType.DMA,
        ],
    )(y_sorted, d1, d2)


@jax.jit
def kernel(hidden_states, gate_w, w_lin, w_v, w_o):
    batch, seq, hdim = hidden_states.shape
    E = gate_w.shape[0]
    x = hidden_states.reshape(-1, hdim)
    T = x.shape[0]
    R = 2 * T
    r_pad = R + E * ROW_TILE

    logits, aux, hist, w1r, w2r = _router(x, gate_w)
    d1 = aux[:, 0].astype(jnp.int32)
    d2 = aux[:, 1].astype(jnp.int32)
    hist_i = hist[0].astype(jnp.int32)

    sorted_x, wsr = _dispatch(x, d1, d2, w1r, w2r, r_pad)
    meta = _gmm_metadata(hist_i, R, E)
    y_sorted = _gmm(sorted_x, w_lin, w_v, w_o, wsr, meta)
    out = _combine(y_sorted, d1, d2)
    return out.reshape(batch, seq, hdim), logits


# speedup vs baseline: 1.1206x; 1.1020x over previous
import jax
import jax.numpy as jnp
from jax import lax
from jax.experimental import pallas as pl
from jax.experimental.pallas import tpu as pltpu
from jax.experimental.pallas import tpu_sc as plsc

NC = 2
NS = 16
NW = NC * NS
LANES = 16
WS_W = 128

ROW_TILE = 256
F_TILE = 2048
CS_TILE = 128


def _sc_mesh():
    return plsc.VectorSubcoreMesh(
        core_axis_name="c", subcore_axis_name="s", num_cores=NC,
        num_subcores=NS)


def _router_body(x_ref, gw_ref, logits_ref, aux_ref, hist_ref, w1r_ref,
                 w2r_ref, cnt_ref, csum_ref):
    x = x_ref[...]
    gw = gw_ref[...]
    logits = lax.dot_general(
        x, gw, (((1,), (1,)), ((), ())), preferred_element_type=jnp.float32)
    T, E = logits.shape
    iota_e = lax.broadcasted_iota(jnp.int32, (T, E), 1)
    m1 = jnp.max(logits, axis=1, keepdims=True)
    i1 = jnp.min(jnp.where(logits == m1, iota_e, E), axis=1, keepdims=True)
    masked = jnp.where(iota_e == i1, -jnp.inf, logits)
    m2 = jnp.max(masked, axis=1, keepdims=True)
    i2 = jnp.min(jnp.where(masked == m2, iota_e, E), axis=1, keepdims=True)
    w1 = jax.nn.sigmoid(m1 - m2)
    w2 = 1.0 - w1
    oh1 = (iota_e == i1).astype(jnp.float32)
    oh2 = (iota_e == i2).astype(jnp.float32)
    cnt_ref[...] = oh1 + oh2

    nt = T // CS_TILE
    iota_r = lax.broadcasted_iota(jnp.int32, (CS_TILE, CS_TILE), 0)
    iota_c = lax.broadcasted_iota(jnp.int32, (CS_TILE, CS_TILE), 1)
    tri = (iota_c < iota_r).astype(jnp.float32)

    def body(k, carry):
        c = cnt_ref[pl.ds(k * CS_TILE, CS_TILE), :]
        excl = lax.dot_general(tri, c, (((1,), (0,)), ((), ())),
                               preferred_element_type=jnp.float32)
        csum_ref[pl.ds(k * CS_TILE, CS_TILE), :] = excl + carry
        return carry + jnp.sum(c, axis=0, keepdims=True)

    hist = lax.fori_loop(0, nt, body, jnp.zeros((1, E), jnp.float32))

    ioe_r = lax.broadcasted_iota(jnp.int32, (E, E), 0)
    ioe_c = lax.broadcasted_iota(jnp.int32, (E, E), 1)
    upper = (ioe_r < ioe_c).astype(jnp.float32)
    tiles_e = jnp.floor((hist + (ROW_TILE - 1)) * (1.0 / ROW_TILE))
    offs = ROW_TILE * lax.dot_general(
        tiles_e, upper, (((1,), (0,)), ((), ())),
        preferred_element_type=jnp.float32)

    csum = csum_ref[...]
    rank1 = jnp.sum(csum * oh1, axis=1, keepdims=True)
    rank2 = jnp.sum(csum * oh2, axis=1, keepdims=True)
    off_b = jnp.broadcast_to(offs, (T, E))
    off1 = jnp.sum(off_b * oh1, axis=1, keepdims=True)
    off2 = jnp.sum(off_b * oh2, axis=1, keepdims=True)
    dest1 = off1 + rank1
    dest2 = off2 + rank2

    logits_ref[...] = logits
    zero = jnp.zeros_like(w1)
    aux_ref[...] = jnp.concatenate(
        [dest1, dest2, w1, w2, i1.astype(jnp.float32),
         i2.astype(jnp.float32), zero, zero], axis=1)
    hist_ref[...] = hist
    w1r_ref[...] = jnp.broadcast_to(w1, (T, WS_W))
    w2r_ref[...] = jnp.broadcast_to(w2, (T, WS_W))


def _router(x, gate_w):
    T, _ = x.shape
    E = gate_w.shape[0]
    return pl.pallas_call(
        _router_body,
        out_shape=(
            jax.ShapeDtypeStruct((T, E), jnp.float32),
            jax.ShapeDtypeStruct((T, 8), jnp.float32),
            jax.ShapeDtypeStruct((1, E), jnp.float32),
            jax.ShapeDtypeStruct((T, WS_W), jnp.float32),
            jax.ShapeDtypeStruct((T, WS_W), jnp.float32),
        ),
        scratch_shapes=[
            pltpu.VMEM((T, E), jnp.float32),
            pltpu.VMEM((T, E), jnp.float32),
        ],
    )(x, gate_w)


def _dispatch_body(x_hbm, d1_hbm, d2_hbm, w1r_hbm, w2r_hbm, sx_hbm, wsr_hbm,
                   i1_v, i2_v, rows_v, w1b_v, w2b_v, sem):
    tpw = rows_v.shape[0]
    wid = lax.axis_index("s") * NC + lax.axis_index("c")
    base = wid * tpw
    pltpu.sync_copy(d1_hbm.at[pl.ds(base, tpw)], i1_v)
    pltpu.sync_copy(d2_hbm.at[pl.ds(base, tpw)], i2_v)
    pltpu.sync_copy(x_hbm.at[pl.ds(base, tpw)], rows_v)
    pltpu.sync_copy(w1r_hbm.at[pl.ds(base, tpw)], w1b_v)
    pltpu.sync_copy(w2r_hbm.at[pl.ds(base, tpw)], w2b_v)
    c1 = pltpu.async_copy(rows_v, sx_hbm.at[i1_v], sem)
    c2 = pltpu.async_copy(rows_v, sx_hbm.at[i2_v], sem)
    c3 = pltpu.async_copy(w1b_v, wsr_hbm.at[i1_v], sem)
    c4 = pltpu.async_copy(w2b_v, wsr_hbm.at[i2_v], sem)
    c1.wait()
    c2.wait()
    c3.wait()
    c4.wait()


def _dispatch(x, d1, d2, w1r, w2r, r_pad):
    T, H = x.shape
    tpw = T // NW
    return pl.kernel(
        _dispatch_body,
        out_type=(
            jax.ShapeDtypeStruct((r_pad, H), jnp.float32),
            jax.ShapeDtypeStruct((r_pad, WS_W), jnp.float32),
        ),
        mesh=_sc_mesh(),
        scratch_types=[
            pltpu.VMEM((tpw,), jnp.int32),
            pltpu.VMEM((tpw,), jnp.int32),
            pltpu.VMEM((tpw, H), jnp.float32),
            pltpu.VMEM((tpw, WS_W), jnp.float32),
            pltpu.VMEM((tpw, WS_W), jnp.float32),
            pltpu.SemaphoreType.DMA,
        ],
    )(x, d1, d2, w1r, w2r)


def _gmm_body(tiles, exps, los, his, firsts, x_ref, wl_ref, wv_ref, wo_ref,
              ws_ref, y_ref):
    s = pl.program_id(0)
    f = pl.program_id(1)

    @pl.when(los[s] < his[s])
    def _body():
        _gmm_step(s, f, los, his, tiles, firsts, x_ref, wl_ref, wv_ref,
                  wo_ref, ws_ref, y_ref)


def _gmm_step(s, f, los, his, tiles, firsts, x_ref, wl_ref, wv_ref, wo_ref,
              ws_ref, y_ref):
    x = x_ref[...]
    wl = wl_ref[0]
    wv = wv_ref[0]
    wo = wo_ref[0]
    dims = (((1,), (1,)), ((), ()))
    a = lax.dot_general(x, wl, dims, preferred_element_type=jnp.float32)
    b = lax.dot_general(x, wv, dims, preferred_element_type=jnp.float32)
    h = (a * jax.nn.sigmoid(a)) * b
    y = lax.dot_general(h, wo, dims, preferred_element_type=jnp.float32)
    TM = y.shape[0]
    gr = tiles[s] * TM + lax.broadcasted_iota(jnp.int32, (TM, 1), 0)
    valid = jnp.logical_and(gr >= los[s], gr < his[s])
    contrib = jnp.where(valid, y * ws_ref[:, 0:1], 0.0)
    is_init = jnp.logical_and(firsts[s] == 1, f == 0)

    @pl.when(is_init)
    def _():
        y_ref[...] = contrib

    @pl.when(jnp.logical_not(is_init))
    def _():
        y_ref[...] += contrib


def _gmm(sorted_x, w_lin, w_v, w_o, wsr, meta):
    R, H = sorted_x.shape
    E, F, _ = w_lin.shape
    tiles, exps, los, his, firsts = meta
    s_max = tiles.shape[0]
    nf = F // F_TILE
    TM = ROW_TILE
    grid_spec = pltpu.PrefetchScalarGridSpec(
        num_scalar_prefetch=5,
        grid=(s_max, nf),
        in_specs=[
            pl.BlockSpec((TM, H), lambda s, f, tiles, *_: (tiles[s], 0)),
            pl.BlockSpec((1, F_TILE, H),
                         lambda s, f, tiles, exps, *_: (exps[s], f, 0)),
            pl.BlockSpec((1, F_TILE, H),
                         lambda s, f, tiles, exps, *_: (exps[s], f, 0)),
            pl.BlockSpec((1, H, F_TILE),
                         lambda s, f, tiles, exps, *_: (exps[s], 0, f)),
            pl.BlockSpec((TM, WS_W), lambda s, f, tiles, *_: (tiles[s], 0)),
        ],
        out_specs=pl.BlockSpec((TM, H), lambda s, f, tiles, *_: (tiles[s], 0)),
    )
    return pl.pallas_call(
        _gmm_body,
        grid_spec=grid_spec,
        out_shape=jax.ShapeDtypeStruct((R, H), jnp.float32),
    )(tiles, exps, los, his, firsts, sorted_x, w_lin, w_v, w_o, wsr)


def _gmm_metadata(hist_i, R, E):
    TM = ROW_TILE
    g = hist_i
    gt = (g + TM - 1) // TM
    vc = jnp.cumsum(gt)
    o_pad = TM * jnp.concatenate([jnp.zeros((1,), vc.dtype), vc[:-1]])
    s_max = R // TM + E
    s_idx = jnp.arange(s_max, dtype=vc.dtype)
    e_s = jnp.searchsorted(vc, s_idx, side="right")
    v_total = vc[-1]
    pad = s_idx >= v_total
    e_c = jnp.clip(e_s, 0, E - 1)
    tile_s = jnp.clip(s_idx, 0, v_total - 1)
    lo_s = jnp.where(pad, 0, o_pad[e_c])
    hi_s = jnp.where(pad, 0, o_pad[e_c] + g[e_c])
    first_s = jnp.logical_not(pad)
    return (tile_s.astype(jnp.int32), e_c.astype(jnp.int32),
            lo_s.astype(jnp.int32), hi_s.astype(jnp.int32),
            first_s.astype(jnp.int32))


def _combine_body(y_hbm, d1_hbm, d2_hbm, out_hbm, i1_v, i2_v, b1_v, b2_v,
                  sem):
    tpw, H = b1_v.shape
    wid = lax.axis_index("s") * NC + lax.axis_index("c")
    base = wid * tpw
    pltpu.sync_copy(d1_hbm.at[pl.ds(base, tpw)], i1_v)
    pltpu.sync_copy(d2_hbm.at[pl.ds(base, tpw)], i2_v)
    c1 = pltpu.async_copy(y_hbm.at[i1_v], b1_v, sem)
    c2 = pltpu.async_copy(y_hbm.at[i2_v], b2_v, sem)
    c1.wait()
    c2.wait()
    npc = H // LANES

    def body(i, _):
        r = i // npc
        c = (i % npc) * LANES
        b1_v[r, pl.ds(c, LANES)] += b2_v[r, pl.ds(c, LANES)]
        return 0

    lax.fori_loop(0, tpw * npc, body, 0)
    pltpu.sync_copy(b1_v, out_hbm.at[pl.ds(base, tpw)])


def _combine(y_sorted, d1, d2):
    R, H = y_sorted.shape
    T = d1.shape[0]
    tpw = T // NW
    return pl.kernel(
        _combine_body,
        out_type=jax.ShapeDtypeStruct((T, H), jnp.float32),
        mesh=_sc_mesh(),
        scratch_types=[
            pltpu.VMEM((tpw,), jnp.int32),
            pltpu.VMEM((tpw,), jnp.int32),
            pltpu.VMEM((tpw, H), jnp.float32),
            pltpu.VMEM((tpw, H), jnp.float32),
            pltpu.SemaphoreType.DMA,
        ],
    )(y_sorted, d1, d2)


@jax.jit
def kernel(hidden_states, gate_w, w_lin, w_v, w_o):
    batch, seq, hdim = hidden_states.shape
    E = gate_w.shape[0]
    x = hidden_states.reshape(-1, hdim)
    T = x.shape[0]
    R = 2 * T
    r_pad = R + E * ROW_TILE

    logits, aux, hist, w1r, w2r = _router(x, gate_w)
    d1 = aux[:, 0].astype(jnp.int32)
    d2 = aux[:, 1].astype(jnp.int32)
    hist_i = hist[0].astype(jnp.int32)

    sorted_x, wsr = _dispatch(x, d1, d2, w1r, w2r, r_pad)
    meta = _gmm_metadata(hist_i, R, E)
    y_sorted = _gmm(sorted_x, w_lin, w_v, w_o, wsr, meta)
    out = _combine(y_sorted, d1, d2)
    return out.reshape(batch, seq, hdim), logits
